# baseline (speedup 1.0000x reference)
import functools

import jax
import jax.numpy as jnp
from jax import lax
from jax.experimental import pallas as pl
from jax.experimental.pallas import tpu as pltpu
from jax.experimental.pallas import tpu_sc as plsc

B = 16384
N_ATTR = 26
H = 128
ENUM_SIZE = 1001
N_NUM = 13
N_STR = 13
EPS = 1e-05

_NC = 2
_NS = 16
_NW = _NC * _NS
_CHUNK = 32
_BPW = B // _NW
_NCHUNK = _BPW // _CHUNK
_L = 16
_HG = H // _L


def _sc_body(xcat_hbm, table_hbm, out_hbm, idxbuf, stage, accb, semg, semo):
    wid = lax.axis_index("s") * _NC + lax.axis_index("c")
    base0 = wid * _BPW

    idx_copies = [
        pltpu.make_async_copy(
            xcat_hbm.at[pl.ds(j * B + base0, _BPW)],
            idxbuf.at[pl.ds(j * _BPW, _BPW)], semo[0])
        for j in range(N_STR)
    ]
    for c in idx_copies:
        c.start()
    for c in idx_copies:
        c.wait()

    def gather_copies(t, p):
        return [pltpu.make_async_copy(
            table_hbm.at[idxbuf.at[pl.ds(j * _BPW + t * _CHUNK, _CHUNK)]],
            stage.at[p, pl.ds(j * _CHUNK, _CHUNK)], semg[p])
            for j in range(N_STR)]

    def out_copy(t, p):
        return pltpu.make_async_copy(
            accb.at[p], out_hbm.at[pl.ds(base0 + t * _CHUNK, _CHUNK)], semo[p])

    def fire(copies):
        for c in copies:
            c.start()

    def wait(copies):
        for c in copies:
            c.wait()

    def accumulate(t, p):
        def _row(r, carry):
            for v in range(_HG):
                sl = pl.ds(v * _L, _L)
                a = stage[p, r, sl]
                for j in range(1, N_STR):
                    a = a + stage[p, j * _CHUNK + r, sl]
                accb[p, r, sl] = a
            return carry

        lax.fori_loop(0, _CHUNK, _row, 0)

    fire(gather_copies(0, 0))

    def step(u, carry):
        t0 = 2 * u
        t1 = t0 + 1
        fire(gather_copies(t1, 1))
        wait(gather_copies(t0, 0))

        @pl.when(u > 0)
        def _():
            out_copy(t0, 0).wait()

        accumulate(t0, 0)
        out_copy(t0, 0).start()

        @pl.when(u < _NCHUNK // 2 - 1)
        def _():
            fire(gather_copies(t0 + 2, 0))

        wait(gather_copies(t1, 1))

        @pl.when(u > 0)
        def _():
            out_copy(t1, 1).wait()

        accumulate(t1, 1)
        out_copy(t1, 1).start()
        return carry

    lax.fori_loop(0, _NCHUNK // 2, step, 0)

    out_copy(_NCHUNK - 2, 0).wait()
    out_copy(_NCHUNK - 1, 1).wait()


def _gather_sum(xcat_flat, table):
    mesh = plsc.VectorSubcoreMesh(core_axis_name="c", subcore_axis_name="s")
    f = pl.kernel(
        _sc_body,
        out_type=jax.ShapeDtypeStruct((B, H), jnp.float32),
        mesh=mesh,
        scratch_types=[
            pltpu.VMEM((N_STR * _BPW,), jnp.int32),
            pltpu.VMEM((2, N_STR * _CHUNK, H), jnp.float32),
            pltpu.VMEM((2, _CHUNK, H), jnp.float32),
            [pltpu.SemaphoreType.DMA, pltpu.SemaphoreType.DMA],
            [pltpu.SemaphoreType.DMA, pltpu.SemaphoreType.DMA],
        ],
    )
    return f(xcat_flat, table)


_NUM_BLK = 1024


def _fin_body(xn_ref, cat_ref, w_ref, b_ref, o_ref):
    xn = xn_ref[...]
    j = lax.broadcasted_iota(jnp.int32, (1, N_NUM), 1).astype(jnp.float32)
    mean = 0.2 * j
    scale = 1.0 / (1.0 + 0.1 * j + EPS)
    s = jax.nn.sigmoid((xn - mean) * scale)
    acc = jnp.dot(s, w_ref[...], preferred_element_type=jnp.float32)
    bias = jnp.sum(b_ref[...], axis=0, keepdims=True)
    o_ref[...] = acc + bias + cat_ref[...]


def _finalize(xnum, catsum, lin_w, lin_b):
    grid = B // _NUM_BLK
    return pl.pallas_call(
        _fin_body,
        grid=(grid,),
        in_specs=[
            pl.BlockSpec((_NUM_BLK, N_NUM), lambda i: (i, 0)),
            pl.BlockSpec((_NUM_BLK, H), lambda i: (i, 0)),
            pl.BlockSpec((N_NUM, H), lambda i: (0, 0)),
            pl.BlockSpec((N_NUM, H), lambda i: (0, 0)),
        ],
        out_specs=pl.BlockSpec((_NUM_BLK, H), lambda i: (i, 0)),
        out_shape=jax.ShapeDtypeStruct((B, H), jnp.float32),
    )(xnum, catsum, lin_w, lin_b)


def kernel(x, lin_w, lin_b, emb):
    offs = (jnp.arange(N_STR, dtype=jnp.int32) * ENUM_SIZE)[:, None]
    xcat_flat = (x[:, 1::2].astype(jnp.int32).T + offs).reshape(-1)
    table = emb.reshape(N_STR * ENUM_SIZE, H)
    catsum = _gather_sum(xcat_flat, table)
    return _finalize(x[:, 0::2], catsum, lin_w, lin_b)


# speedup vs baseline: 1.0689x; 1.0689x over previous
import functools

import jax
import jax.numpy as jnp
from jax import lax
from jax.experimental import pallas as pl
from jax.experimental.pallas import tpu as pltpu
from jax.experimental.pallas import tpu_sc as plsc

B = 16384
N_ATTR = 26
H = 128
ENUM_SIZE = 1001
N_NUM = 13
N_STR = 13
EPS = 1e-05

_NC = 2
_NS = 16
_NW = _NC * _NS
_CHUNK = 32
_BPW = B // _NW
_NCHUNK = _BPW // _CHUNK
_L = 16
_HG = H // _L


def _sc_body(xcat_hbm, table_hbm, out_hbm, idxbuf, stage, accb, semg, semo):
    wid = lax.axis_index("s") * _NC + lax.axis_index("c")
    base0 = wid * _BPW

    idx_copies = [
        pltpu.make_async_copy(
            xcat_hbm.at[pl.ds(j * B + base0, _BPW)],
            idxbuf.at[pl.ds(j * _BPW, _BPW)], semo[0])
        for j in range(N_STR)
    ]
    for c in idx_copies:
        c.start()
    for c in idx_copies:
        c.wait()

    def gather_copies(t, p):
        return [pltpu.make_async_copy(
            table_hbm.at[idxbuf.at[pl.ds(j * _BPW + t * _CHUNK, _CHUNK)]],
            stage.at[p, pl.ds(j * _CHUNK, _CHUNK)], semg[p])
            for j in range(N_STR)]

    def out_copy(t, p):
        return pltpu.make_async_copy(
            accb.at[p], out_hbm.at[pl.ds(base0 + t * _CHUNK, _CHUNK)], semo[p])

    def fire(copies):
        for c in copies:
            c.start()

    def wait(copies):
        for c in copies:
            c.wait()

    def accumulate(t, p):
        def _row(r, carry):
            for v in range(_HG):
                sl = pl.ds(v * _L, _L)
                a = stage[p, r, sl]
                for j in range(1, N_STR):
                    a = a + stage[p, j * _CHUNK + r, sl]
                accb[p, r, sl] = a
            return carry

        lax.fori_loop(0, _CHUNK, _row, 0)

    fire(gather_copies(0, 0))

    def step(u, carry):
        t0 = 2 * u
        t1 = t0 + 1
        fire(gather_copies(t1, 1))
        wait(gather_copies(t0, 0))

        @pl.when(u > 0)
        def _():
            out_copy(t0, 0).wait()

        accumulate(t0, 0)
        out_copy(t0, 0).start()

        @pl.when(u < _NCHUNK // 2 - 1)
        def _():
            fire(gather_copies(t0 + 2, 0))

        wait(gather_copies(t1, 1))

        @pl.when(u > 0)
        def _():
            out_copy(t1, 1).wait()

        accumulate(t1, 1)
        out_copy(t1, 1).start()
        return carry

    lax.fori_loop(0, _NCHUNK // 2, step, 0)

    out_copy(_NCHUNK - 2, 0).wait()
    out_copy(_NCHUNK - 1, 1).wait()


def _gather_sum(xcat_flat, table):
    mesh = plsc.VectorSubcoreMesh(core_axis_name="c", subcore_axis_name="s")
    f = pl.kernel(
        _sc_body,
        out_type=jax.ShapeDtypeStruct((B, H), jnp.float32),
        mesh=mesh,
        scratch_types=[
            pltpu.VMEM((N_STR * _BPW,), jnp.int32),
            pltpu.VMEM((2, N_STR * _CHUNK, H), jnp.float32),
            pltpu.VMEM((2, _CHUNK, H), jnp.float32),
            [pltpu.SemaphoreType.DMA, pltpu.SemaphoreType.DMA],
            [pltpu.SemaphoreType.DMA, pltpu.SemaphoreType.DMA],
        ],
    )
    return f(xcat_flat, table)


_NUM_BLK = 4096


def _fin_body(xn_ref, cat_ref, w_ref, b_ref, o_ref):
    xn = xn_ref[...]
    j = lax.broadcasted_iota(jnp.int32, (1, N_NUM), 1).astype(jnp.float32)
    mean = 0.2 * j
    scale = 1.0 / (1.0 + 0.1 * j + EPS)
    s = jax.nn.sigmoid((xn - mean) * scale)
    acc = jnp.dot(s, w_ref[...], preferred_element_type=jnp.float32)
    bias = jnp.sum(b_ref[...], axis=0, keepdims=True)
    o_ref[...] = acc + bias + cat_ref[...]


def _finalize(xnum, catsum, lin_w, lin_b):
    grid = B // _NUM_BLK
    return pl.pallas_call(
        _fin_body,
        grid=(grid,),
        in_specs=[
            pl.BlockSpec((_NUM_BLK, N_NUM), lambda i: (i, 0)),
            pl.BlockSpec((_NUM_BLK, H), lambda i: (i, 0)),
            pl.BlockSpec((N_NUM, H), lambda i: (0, 0)),
            pl.BlockSpec((N_NUM, H), lambda i: (0, 0)),
        ],
        out_specs=pl.BlockSpec((_NUM_BLK, H), lambda i: (i, 0)),
        out_shape=jax.ShapeDtypeStruct((B, H), jnp.float32),
    )(xnum, catsum, lin_w, lin_b)


def kernel(x, lin_w, lin_b, emb):
    offs = (jnp.arange(N_STR, dtype=jnp.int32) * ENUM_SIZE)[:, None]
    xcat_flat = (x[:, 1::2].astype(jnp.int32).T + offs).reshape(-1)
    table = emb.reshape(N_STR * ENUM_SIZE, H)
    catsum = _gather_sum(xcat_flat, table)
    return _finalize(x[:, 0::2], catsum, lin_w, lin_b)


# speedup vs baseline: 1.0796x; 1.0100x over previous
import functools

import jax
import jax.numpy as jnp
from jax import lax
from jax.experimental import pallas as pl
from jax.experimental.pallas import tpu as pltpu
from jax.experimental.pallas import tpu_sc as plsc

B = 16384
N_ATTR = 26
H = 128
ENUM_SIZE = 1001
N_NUM = 13
N_STR = 13
EPS = 1e-05

_NC = 2
_NS = 16
_NW = _NC * _NS
_CHUNK = 32
_BPW = B // _NW
_NCHUNK = _BPW // _CHUNK
_L = 16
_HG = H // _L


def _sc_body(xcat_hbm, table_hbm, out_hbm, idxbuf, stage, accb, semg, semo):
    wid = lax.axis_index("s") * _NC + lax.axis_index("c")
    base0 = wid * _BPW

    idx_copies = [
        pltpu.make_async_copy(
            xcat_hbm.at[pl.ds(j * B + base0, _BPW)],
            idxbuf.at[pl.ds(j * _BPW, _BPW)], semo[0])
        for j in range(N_STR)
    ]
    for c in idx_copies:
        c.start()
    for c in idx_copies:
        c.wait()

    def gather_copies(t, p):
        return [pltpu.make_async_copy(
            table_hbm.at[idxbuf.at[pl.ds(j * _BPW + t * _CHUNK, _CHUNK)]],
            stage.at[p, pl.ds(j * _CHUNK, _CHUNK)], semg[p])
            for j in range(N_STR)]

    def out_copy(t, p):
        return pltpu.make_async_copy(
            accb.at[p], out_hbm.at[pl.ds(base0 + t * _CHUNK, _CHUNK)], semo[p])

    def fire(copies):
        for c in copies:
            c.start()

    def wait(copies):
        for c in copies:
            c.wait()

    def accumulate(t, p):
        def _row(r, carry):
            for v in range(_HG):
                sl = pl.ds(v * _L, _L)
                a = stage[p, r, sl]
                for j in range(1, N_STR):
                    a = a + stage[p, j * _CHUNK + r, sl]
                accb[p, r, sl] = a
            return carry

        lax.fori_loop(0, _CHUNK, _row, 0)

    fire(gather_copies(0, 0))

    def step(u, carry):
        t0 = 2 * u
        t1 = t0 + 1
        fire(gather_copies(t1, 1))
        wait(gather_copies(t0, 0))

        @pl.when(u > 0)
        def _():
            out_copy(t0, 0).wait()

        accumulate(t0, 0)
        out_copy(t0, 0).start()

        @pl.when(u < _NCHUNK // 2 - 1)
        def _():
            fire(gather_copies(t0 + 2, 0))

        wait(gather_copies(t1, 1))

        @pl.when(u > 0)
        def _():
            out_copy(t1, 1).wait()

        accumulate(t1, 1)
        out_copy(t1, 1).start()
        return carry

    lax.fori_loop(0, _NCHUNK // 2, step, 0)

    out_copy(_NCHUNK - 2, 0).wait()
    out_copy(_NCHUNK - 1, 1).wait()


def _gather_sum(xcat_flat, table):
    mesh = plsc.VectorSubcoreMesh(core_axis_name="c", subcore_axis_name="s")
    f = pl.kernel(
        _sc_body,
        out_type=jax.ShapeDtypeStruct((B, H), jnp.float32),
        mesh=mesh,
        scratch_types=[
            pltpu.VMEM((N_STR * _BPW,), jnp.int32),
            pltpu.VMEM((2, N_STR * _CHUNK, H), jnp.float32),
            pltpu.VMEM((2, _CHUNK, H), jnp.float32),
            [pltpu.SemaphoreType.DMA, pltpu.SemaphoreType.DMA],
            [pltpu.SemaphoreType.DMA, pltpu.SemaphoreType.DMA],
        ],
    )
    return f(xcat_flat, table)


_NUM_BLK = 8192


def _fin_body(xn_ref, cat_ref, w_ref, b_ref, o_ref):
    xn = xn_ref[...]
    j = lax.broadcasted_iota(jnp.int32, (1, N_NUM), 1).astype(jnp.float32)
    mean = 0.2 * j
    scale = 1.0 / (1.0 + 0.1 * j + EPS)
    s = jax.nn.sigmoid((xn - mean) * scale)
    acc = jnp.dot(s, w_ref[...], preferred_element_type=jnp.float32)
    bias = jnp.sum(b_ref[...], axis=0, keepdims=True)
    o_ref[...] = acc + bias + cat_ref[...]


def _finalize(xnum, catsum, lin_w, lin_b):
    grid = B // _NUM_BLK
    return pl.pallas_call(
        _fin_body,
        grid=(grid,),
        in_specs=[
            pl.BlockSpec((_NUM_BLK, N_NUM), lambda i: (i, 0)),
            pl.BlockSpec((_NUM_BLK, H), lambda i: (i, 0)),
            pl.BlockSpec((N_NUM, H), lambda i: (0, 0)),
            pl.BlockSpec((N_NUM, H), lambda i: (0, 0)),
        ],
        out_specs=pl.BlockSpec((_NUM_BLK, H), lambda i: (i, 0)),
        out_shape=jax.ShapeDtypeStruct((B, H), jnp.float32),
    )(xnum, catsum, lin_w, lin_b)


def kernel(x, lin_w, lin_b, emb):
    offs = (jnp.arange(N_STR, dtype=jnp.int32) * ENUM_SIZE)[:, None]
    xcat_flat = (x[:, 1::2].astype(jnp.int32).T + offs).reshape(-1)
    table = emb.reshape(N_STR * ENUM_SIZE, H)
    catsum = _gather_sum(xcat_flat, table)
    return _finalize(x[:, 0::2], catsum, lin_w, lin_b)


# speedup vs baseline: 1.0799x; 1.0003x over previous
import jax
import jax.numpy as jnp
from jax import lax
from jax.experimental import pallas as pl
from jax.experimental.pallas import tpu as pltpu
from jax.experimental.pallas import tpu_sc as plsc

B = 16384
N_ATTR = 26
H = 128
ENUM_SIZE = 1001
N_NUM = 13
N_STR = 13
EPS = 1e-05

_NC = 2
_NS = 16
_NW = _NC * _NS
_CHUNK = 32
_BPW = B // _NW
_NCHUNK = _BPW // _CHUNK
_L = 16
_HG = H // _L


def _sc_body(xcat_hbm, table_hbm, out_hbm, idxbuf, stage, accb, semg, semo):
    wid = lax.axis_index("s") * _NC + lax.axis_index("c")
    base0 = wid * _BPW

    idx_copies = [
        pltpu.make_async_copy(
            xcat_hbm.at[pl.ds(j * B + base0, _BPW)],
            idxbuf.at[pl.ds(j * _BPW, _BPW)], semo[0])
        for j in range(N_STR)
    ]
    for c in idx_copies:
        c.start()
    for c in idx_copies:
        c.wait()

    def gather_copies(t, p):
        return [pltpu.make_async_copy(
            table_hbm.at[idxbuf.at[pl.ds(j * _BPW + t * _CHUNK, _CHUNK)]],
            stage.at[p, pl.ds(j * _CHUNK, _CHUNK)], semg[p])
            for j in range(N_STR)]

    def out_copy(t, p):
        return pltpu.make_async_copy(
            accb.at[p], out_hbm.at[pl.ds(base0 + t * _CHUNK, _CHUNK)], semo[p])

    def fire(copies):
        for c in copies:
            c.start()

    def wait(copies):
        for c in copies:
            c.wait()

    def accumulate(t, p):
        def _row(r, carry):
            for v in range(_HG):
                sl = pl.ds(v * _L, _L)
                a = stage[p, r, sl]
                for j in range(1, N_STR):
                    a = a + stage[p, j * _CHUNK + r, sl]
                accb[p, r, sl] = a
            return carry

        lax.fori_loop(0, _CHUNK, _row, 0)

    fire(gather_copies(0, 0))

    def step(u, carry):
        t0 = 2 * u
        t1 = t0 + 1
        fire(gather_copies(t1, 1))
        wait(gather_copies(t0, 0))

        @pl.when(u > 0)
        def _():
            out_copy(t0, 0).wait()

        accumulate(t0, 0)
        out_copy(t0, 0).start()

        @pl.when(u < _NCHUNK // 2 - 1)
        def _():
            fire(gather_copies(t0 + 2, 0))

        wait(gather_copies(t1, 1))

        @pl.when(u > 0)
        def _():
            out_copy(t1, 1).wait()

        accumulate(t1, 1)
        out_copy(t1, 1).start()
        return carry

    lax.fori_loop(0, _NCHUNK // 2, step, 0)

    out_copy(_NCHUNK - 2, 0).wait()
    out_copy(_NCHUNK - 1, 1).wait()


def _gather_sum(xcat_flat, table):
    mesh = plsc.VectorSubcoreMesh(core_axis_name="c", subcore_axis_name="s")
    f = pl.kernel(
        _sc_body,
        out_type=jax.ShapeDtypeStruct((B, H), jnp.float32),
        mesh=mesh,
        scratch_types=[
            pltpu.VMEM((N_STR * _BPW,), jnp.int32),
            pltpu.VMEM((2, N_STR * _CHUNK, H), jnp.float32),
            pltpu.VMEM((2, _CHUNK, H), jnp.float32),
            [pltpu.SemaphoreType.DMA, pltpu.SemaphoreType.DMA],
            [pltpu.SemaphoreType.DMA, pltpu.SemaphoreType.DMA],
        ],
    )
    return f(xcat_flat, table)


_NUM_BLK = 8192


def _fin_body(xn_ref, cat_ref, w_ref, b_ref, o_ref):
    xn = xn_ref[...]
    j = lax.broadcasted_iota(jnp.int32, (1, N_NUM), 1).astype(jnp.float32)
    mean = 0.2 * j
    scale = 1.0 / (1.0 + 0.1 * j + EPS)
    s = jax.nn.sigmoid((xn - mean) * scale)
    acc = jnp.dot(s, w_ref[...], preferred_element_type=jnp.float32)
    bias = jnp.sum(b_ref[...], axis=0, keepdims=True)
    o_ref[...] = acc + bias + cat_ref[...]


def _finalize(xnum, catsum, lin_w, lin_b):
    grid = B // _NUM_BLK
    return pl.pallas_call(
        _fin_body,
        grid=(grid,),
        in_specs=[
            pl.BlockSpec((_NUM_BLK, N_NUM), lambda i: (i, 0)),
            pl.BlockSpec((_NUM_BLK, H), lambda i: (i, 0)),
            pl.BlockSpec((N_NUM, H), lambda i: (0, 0)),
            pl.BlockSpec((N_NUM, H), lambda i: (0, 0)),
        ],
        out_specs=pl.BlockSpec((_NUM_BLK, H), lambda i: (i, 0)),
        out_shape=jax.ShapeDtypeStruct((B, H), jnp.float32),
    )(xnum, catsum, lin_w, lin_b)


def kernel(x, lin_w, lin_b, emb):
    offs = (jnp.arange(N_STR, dtype=jnp.int32) * ENUM_SIZE)[:, None]
    xcat_flat = (x[:, 1::2].astype(jnp.int32).T + offs).reshape(-1)
    table = emb.reshape(N_STR * ENUM_SIZE, H)
    catsum = _gather_sum(xcat_flat, table)
    return _finalize(x[:, 0::2], catsum, lin_w, lin_b)
